# baseline (speedup 1.0000x reference)
import functools

import jax
import jax.numpy as jnp
from jax import lax
from jax.experimental import pallas as pl
from jax.experimental.pallas import tpu as pltpu
from jax.experimental.pallas import tpu_sc as plsc

_B = 16384
_D = 64
_NC = 2
_NS = 16
_NW = _NC * _NS
_BPW = _B // _NW
_C = 16
_NCHUNK = _BPW // _C
_NPAIR = _NCHUNK // 2
_L = 16


@functools.cache
def _gather_kernel():
    mesh = plsc.VectorSubcoreMesh(
        core_axis_name="c", subcore_axis_name="s",
        num_cores=_NC, num_subcores=_NS,
    )

    @functools.partial(
        pl.kernel,
        out_type=jax.ShapeDtypeStruct((_B, _D), jnp.float32),
        mesh=mesh,
        scratch_types=[
            pltpu.VMEM((_BPW,), jnp.int32),
            pltpu.VMEM((_C, 8, _D), jnp.float32),
            pltpu.VMEM((_C, 8, _D), jnp.float32),
            pltpu.VMEM((_BPW, _D), jnp.float32),
            pltpu.SemaphoreType.DMA,
            pltpu.SemaphoreType.DMA,
            pltpu.SemaphoreType.DMA,
        ],
    )
    def body(idx_hbm, table_hbm, out_hbm, lab_v, tiles0, tiles1,
             out_v, sem_in, sem0, sem1):
        wid = lax.axis_index("s") * _NC + lax.axis_index("c")
        base = wid * _BPW
        pltpu.async_copy(idx_hbm.at[pl.ds(base, _BPW)], lab_v, sem_in).wait()

        def fire(ch, buf, sem):
            lab = lab_v[pl.ds(ch * _C, _C)]
            for e in range(_C):
                row0 = pl.multiple_of(
                    lax.shift_right_logical(lab[e], 3) * 8, 8
                )
                pltpu.async_copy(
                    table_hbm.at[pl.ds(row0, 8)], buf.at[e], sem
                )

        def drain(buf, sem):
            for e in range(_C):
                pltpu.make_async_copy(
                    table_hbm.at[pl.ds(0, 8)], buf.at[e], sem
                ).wait()

        def extract(ch, buf):
            off = ch * _C
            lab = lab_v[pl.ds(off, _C)]
            for e in range(_C):
                r = lax.bitwise_and(lab[e], 7)
                for c in range(_D // _L):
                    out_v[off + e, pl.ds(c * _L, _L)] = (
                        buf[e, r, pl.ds(c * _L, _L)]
                    )

        fire(0, tiles0, sem0)

        def do_pair(p, _):
            fire(2 * p + 1, tiles1, sem1)
            drain(tiles0, sem0)
            extract(2 * p, tiles0)

            @pl.when(p < _NPAIR - 1)
            def _():
                fire(2 * p + 2, tiles0, sem0)

            drain(tiles1, sem1)
            extract(2 * p + 1, tiles1)
            return ()

        lax.fori_loop(0, _NPAIR, do_pair, ())
        pltpu.sync_copy(out_v, out_hbm.at[pl.ds(base, _BPW)])

    return body


def kernel(class_label, table, uncond_table):
    del uncond_table
    idx = class_label.astype(jnp.int32)
    out = _gather_kernel()(idx, table)
    return out.reshape(_B, 1, _D)


# speedup vs baseline: 1.4657x; 1.4657x over previous
import functools

import jax
import jax.numpy as jnp
from jax import lax
from jax.experimental import pallas as pl
from jax.experimental.pallas import tpu as pltpu
from jax.experimental.pallas import tpu_sc as plsc

_B = 16384
_D = 64
_NC = 2
_NS = 16
_NW = _NC * _NS
_BPW = _B // _NW
_C = 8
_NGRP = _BPW // 16
_L = 16


@functools.cache
def _gather_kernel():
    mesh = plsc.VectorSubcoreMesh(
        core_axis_name="c", subcore_axis_name="s",
        num_cores=_NC, num_subcores=_NS,
    )

    block_types = [pltpu.VMEM((_D, 128), jnp.float32) for _ in range(_C)]

    @functools.partial(
        pl.kernel,
        out_type=jax.ShapeDtypeStruct((_D, _B), jnp.float32),
        mesh=mesh,
        scratch_types=[
            pltpu.VMEM((_BPW,), jnp.int32),
            *block_types,
            pltpu.VMEM((_D, _BPW), jnp.float32),
            pltpu.SemaphoreType.DMA,
            pltpu.SemaphoreType.DMA,
        ],
        compiler_params=pltpu.CompilerParams(needs_layout_passes=False),
    )
    def body(idx_hbm, tableT_hbm, outT_hbm, lab_v, *rest):
        blocks = rest[:_C]
        outT_v, sem_in, sem_g = rest[_C], rest[_C + 1], rest[_C + 2]
        wid = lax.axis_index("s") * _NC + lax.axis_index("c")
        base = wid * _BPW
        pltpu.async_copy(idx_hbm.at[pl.ds(base, _BPW)], lab_v, sem_in).wait()

        lane = lax.iota(jnp.int32, _L)
        zeros = jnp.zeros((_L,), jnp.int32)

        def do_group(g, _):
            lab16 = lab_v[pl.ds(g * _L, _L)]
            for half in range(2):
                off = g * _L + half * _C
                for e in range(_C):
                    blk0 = pl.multiple_of(
                        lax.bitwise_and(lab16[half * _C + e], -128), 128
                    )
                    pltpu.async_copy(
                        tableT_hbm.at[:, pl.ds(blk0, 128)],
                        blocks[e],
                        sem_g,
                    )
                for e in range(_C):
                    pltpu.make_async_copy(
                        tableT_hbm.at[:, pl.ds(0, 128)], blocks[e], sem_g
                    ).wait()
                for e in range(_C):
                    l_in = lax.bitwise_and(lab16[half * _C + e], 127)
                    l_vec = zeros + l_in
                    p_vec = zeros + (off + e)
                    for c in range(_D // _L):
                        val = plsc.load_gather(
                            blocks[e], [c * _L + lane, l_vec]
                        )
                        plsc.store_scatter(
                            outT_v, [c * _L + lane, p_vec], val
                        )
            return ()

        lax.fori_loop(0, _NGRP, do_group, ())
        pltpu.sync_copy(outT_v, outT_hbm.at[:, pl.ds(base, _BPW)])

    return body


def kernel(class_label, table, uncond_table):
    del uncond_table
    idx = class_label.astype(jnp.int32)
    outT = _gather_kernel()(idx, table.T)
    return outT.T.reshape(_B, 1, _D)


# speedup vs baseline: 1.5851x; 1.0815x over previous
import functools

import jax
import jax.numpy as jnp
from jax import lax
from jax.experimental import pallas as pl
from jax.experimental.pallas import tpu as pltpu
from jax.experimental.pallas import tpu_sc as plsc

_B = 16384
_D = 64
_NC = 2
_NS = 16
_NW = _NC * _NS
_BPW = _B // _NW
_C = 4
_NGRP = _BPW // 16
_L = 16


@functools.cache
def _gather_kernel():
    mesh = plsc.VectorSubcoreMesh(
        core_axis_name="c", subcore_axis_name="s",
        num_cores=_NC, num_subcores=_NS,
    )

    block_types = [pltpu.VMEM((_D, 128), jnp.float32) for _ in range(2 * _C)]

    @functools.partial(
        pl.kernel,
        out_type=jax.ShapeDtypeStruct((_D, _B), jnp.float32),
        mesh=mesh,
        scratch_types=[
            pltpu.VMEM((_BPW,), jnp.int32),
            *block_types,
            pltpu.VMEM((_D, _BPW), jnp.float32),
            pltpu.SemaphoreType.DMA,
            pltpu.SemaphoreType.DMA,
            pltpu.SemaphoreType.DMA,
        ],
        compiler_params=pltpu.CompilerParams(needs_layout_passes=False),
    )
    def body(idx_hbm, tableT_hbm, outT_hbm, lab_v, *rest):
        buf_a = rest[:_C]
        buf_b = rest[_C:2 * _C]
        outT_v = rest[2 * _C]
        sem_in, sem_a, sem_b = rest[2 * _C + 1:]
        wid = lax.axis_index("s") * _NC + lax.axis_index("c")
        base = wid * _BPW
        pltpu.async_copy(idx_hbm.at[pl.ds(base, _BPW)], lab_v, sem_in).wait()

        lane = lax.iota(jnp.int32, _L)
        zeros = jnp.zeros((_L,), jnp.int32)

        def fire(lab16, lbase, bufs, sem):
            for e in range(_C):
                blk0 = pl.multiple_of(
                    lax.bitwise_and(lab16[lbase + e], -128), 128
                )
                pltpu.async_copy(
                    tableT_hbm.at[:, pl.ds(blk0, 128)], bufs[e], sem
                )

        def drain(bufs, sem):
            for e in range(_C):
                pltpu.make_async_copy(
                    tableT_hbm.at[:, pl.ds(0, 128)], bufs[e], sem
                ).wait()

        def extract(lab16, lbase, off, bufs):
            for e in range(_C):
                l_vec = zeros + lax.bitwise_and(lab16[lbase + e], 127)
                p_vec = zeros + (off + e)
                for c in range(_D // _L):
                    val = plsc.load_gather(bufs[e], [c * _L + lane, l_vec])
                    plsc.store_scatter(outT_v, [c * _L + lane, p_vec], val)

        lab0 = lab_v[pl.ds(0, _L)]
        fire(lab0, 0, buf_a, sem_a)

        def do_group(g, _):
            lab16 = lab_v[pl.ds(g * _L, _L)]
            off = g * _L
            fire(lab16, _C, buf_b, sem_b)
            drain(buf_a, sem_a)
            extract(lab16, 0, off, buf_a)

            fire(lab16, 2 * _C, buf_a, sem_a)
            drain(buf_b, sem_b)
            extract(lab16, _C, off + _C, buf_b)

            fire(lab16, 3 * _C, buf_b, sem_b)
            drain(buf_a, sem_a)
            extract(lab16, 2 * _C, off + 2 * _C, buf_a)

            @pl.when(g < _NGRP - 1)
            def _():
                lab_n = lab_v[pl.ds((g + 1) * _L, _L)]
                fire(lab_n, 0, buf_a, sem_a)

            drain(buf_b, sem_b)
            extract(lab16, 3 * _C, off + 3 * _C, buf_b)
            return ()

        lax.fori_loop(0, _NGRP, do_group, ())
        pltpu.sync_copy(outT_v, outT_hbm.at[:, pl.ds(base, _BPW)])

    return body


def kernel(class_label, table, uncond_table):
    del uncond_table
    idx = class_label.astype(jnp.int32)
    outT = _gather_kernel()(idx, table.T)
    return outT.T.reshape(_B, 1, _D)
